# baseline (speedup 1.0000x reference)
import jax
import jax.numpy as jnp
from jax.experimental import pallas as pl
from jax.experimental.pallas import tpu as pltpu


def _split_bf16(a):
    hi = a.astype(jnp.bfloat16)
    lo = (a - hi.astype(jnp.float32)).astype(jnp.bfloat16)
    return hi, lo


def _gate_kernel(x_ref, z_ref, w1h_ref, w1l_ref, w2h_ref, w2l_ref, b_ref,
                 out_ref):
    xh, xl = _split_bf16(x_ref[...])
    zh, zl = _split_bf16(z_ref[...])
    f32 = jnp.float32
    logits = jnp.dot(xh, w1h_ref[...], preferred_element_type=f32)
    logits += jnp.dot(xh, w1l_ref[...], preferred_element_type=f32)
    logits += jnp.dot(xl, w1h_ref[...], preferred_element_type=f32)
    logits += jnp.dot(zh, w2h_ref[...], preferred_element_type=f32)
    logits += jnp.dot(zh, w2l_ref[...], preferred_element_type=f32)
    logits += jnp.dot(zl, w2h_ref[...], preferred_element_type=f32)
    logits += b_ref[...]
    m = jnp.max(logits, axis=1, keepdims=True)
    e = jnp.exp(logits - m)
    out_ref[...] = e / jnp.sum(e, axis=1, keepdims=True)


def kernel(x, z, W, b):
    n_tokens, dx = x.shape
    dz = z.shape[1]
    num_experts = W.shape[1]
    w1h, w1l = _split_bf16(W[:dx])
    w2h, w2l = _split_bf16(W[dx:])
    b2 = b.reshape(1, num_experts)

    block = 512
    grid = (n_tokens // block,)

    return pl.pallas_call(
        _gate_kernel,
        grid=grid,
        in_specs=[
            pl.BlockSpec((block, dx), lambda i: (i, 0)),
            pl.BlockSpec((block, dz), lambda i: (i, 0)),
            pl.BlockSpec((dx, num_experts), lambda i: (0, 0)),
            pl.BlockSpec((dx, num_experts), lambda i: (0, 0)),
            pl.BlockSpec((dz, num_experts), lambda i: (0, 0)),
            pl.BlockSpec((dz, num_experts), lambda i: (0, 0)),
            pl.BlockSpec((1, num_experts), lambda i: (0, 0)),
        ],
        out_specs=pl.BlockSpec((block, num_experts), lambda i: (i, 0)),
        out_shape=jax.ShapeDtypeStruct((n_tokens, num_experts), jnp.float32),
        compiler_params=pltpu.CompilerParams(
            dimension_semantics=("parallel",),
        ),
    )(x, z, w1h, w1l, w2h, w2l, b2)


# speedup vs baseline: 1.1656x; 1.1656x over previous
import jax
import jax.numpy as jnp
from jax.experimental import pallas as pl
from jax.experimental.pallas import tpu as pltpu


def _gate_kernel(x_ref, z_ref, w1h_ref, w1l_ref, w2h_ref, w2l_ref, b_ref,
                 out_ref):
    f32 = jnp.float32
    xh = x_ref[...].astype(jnp.bfloat16)
    zh = z_ref[...].astype(jnp.bfloat16)
    logits = jnp.dot(xh, w1h_ref[...], preferred_element_type=f32)
    logits += jnp.dot(xh, w1l_ref[...], preferred_element_type=f32)
    logits += jnp.dot(zh, w2h_ref[...], preferred_element_type=f32)
    logits += jnp.dot(zh, w2l_ref[...], preferred_element_type=f32)
    logits += b_ref[...]
    m = jnp.max(logits, axis=1, keepdims=True)
    e = jnp.exp(logits - m)
    out_ref[...] = e / jnp.sum(e, axis=1, keepdims=True)


def kernel(x, z, W, b):
    n_tokens, dx = x.shape
    dz = z.shape[1]
    num_experts = W.shape[1]
    w1 = W[:dx]
    w2 = W[dx:]
    w1h = w1.astype(jnp.bfloat16)
    w1l = (w1 - w1h.astype(jnp.float32)).astype(jnp.bfloat16)
    w2h = w2.astype(jnp.bfloat16)
    w2l = (w2 - w2h.astype(jnp.float32)).astype(jnp.bfloat16)
    b2 = b.reshape(1, num_experts)

    block = 512
    grid = (n_tokens // block,)

    return pl.pallas_call(
        _gate_kernel,
        grid=grid,
        in_specs=[
            pl.BlockSpec((block, dx), lambda i: (i, 0)),
            pl.BlockSpec((block, dz), lambda i: (i, 0)),
            pl.BlockSpec((dx, num_experts), lambda i: (0, 0)),
            pl.BlockSpec((dx, num_experts), lambda i: (0, 0)),
            pl.BlockSpec((dz, num_experts), lambda i: (0, 0)),
            pl.BlockSpec((dz, num_experts), lambda i: (0, 0)),
            pl.BlockSpec((1, num_experts), lambda i: (0, 0)),
        ],
        out_specs=pl.BlockSpec((block, num_experts), lambda i: (i, 0)),
        out_shape=jax.ShapeDtypeStruct((n_tokens, num_experts), jnp.float32),
        compiler_params=pltpu.CompilerParams(
            dimension_semantics=("parallel",),
        ),
    )(x, z, w1h, w1l, w2h, w2l, b2)


# speedup vs baseline: 1.2638x; 1.0843x over previous
import jax
import jax.numpy as jnp
from jax.experimental import pallas as pl
from jax.experimental.pallas import tpu as pltpu


def _gate_kernel(x_ref, z_ref, w1_ref, w2_ref, b_ref, out_ref,
                 xh_ref, zh_ref):
    f32 = jnp.float32
    bf16 = jnp.bfloat16
    n = out_ref.shape[1]
    xh_ref[...] = x_ref[...].astype(bf16)
    zh_ref[...] = z_ref[...].astype(bf16)
    p = jnp.dot(xh_ref[...], w1_ref[...], preferred_element_type=f32)
    p += jnp.dot(zh_ref[...], w2_ref[...], preferred_element_type=f32)
    logits = p[:, :n] + p[:, n:] + b_ref[...]
    m = jnp.max(logits, axis=1, keepdims=True)
    e = jnp.exp(logits - m)
    out_ref[...] = e / jnp.sum(e, axis=1, keepdims=True)


def _split_cat(w):
    hi = w.astype(jnp.bfloat16)
    lo = (w - hi.astype(jnp.float32)).astype(jnp.bfloat16)
    return jnp.concatenate([hi, lo], axis=1)


def kernel(x, z, W, b):
    n_tokens, dx = x.shape
    dz = z.shape[1]
    num_experts = W.shape[1]
    w1 = _split_cat(W[:dx])
    w2 = _split_cat(W[dx:])
    b2 = b.reshape(1, num_experts)

    block = 512
    grid = (n_tokens // block,)

    return pl.pallas_call(
        _gate_kernel,
        grid=grid,
        in_specs=[
            pl.BlockSpec((block, dx), lambda i: (i, 0)),
            pl.BlockSpec((block, dz), lambda i: (i, 0)),
            pl.BlockSpec((dx, 2 * num_experts), lambda i: (0, 0)),
            pl.BlockSpec((dz, 2 * num_experts), lambda i: (0, 0)),
            pl.BlockSpec((1, num_experts), lambda i: (0, 0)),
        ],
        out_specs=pl.BlockSpec((block, num_experts), lambda i: (i, 0)),
        out_shape=jax.ShapeDtypeStruct((n_tokens, num_experts), jnp.float32),
        scratch_shapes=[
            pltpu.VMEM((block, dx), jnp.bfloat16),
            pltpu.VMEM((block, dz), jnp.bfloat16),
        ],
        compiler_params=pltpu.CompilerParams(
            dimension_semantics=("parallel",),
        ),
    )(x, z, w1, w2, b2)


# speedup vs baseline: 1.3094x; 1.0361x over previous
import jax
import jax.numpy as jnp
from jax.experimental import pallas as pl
from jax.experimental.pallas import tpu as pltpu


def _gate_kernel(x_ref, z_ref, w1_ref, w2_ref, b_ref, out_ref):
    f32 = jnp.float32
    bf16 = jnp.bfloat16
    n = out_ref.shape[1]
    xh = x_ref[...].astype(bf16)
    zh = z_ref[...].astype(bf16)
    p = jnp.dot(xh, w1_ref[...], preferred_element_type=f32)
    p += jnp.dot(zh, w2_ref[...], preferred_element_type=f32)
    logits = p[:, :n] + p[:, n:] + b_ref[...]
    m = jnp.max(logits, axis=1, keepdims=True)
    e = jnp.exp(logits - m)
    out_ref[...] = e / jnp.sum(e, axis=1, keepdims=True)


def _split_cat(w):
    hi = w.astype(jnp.bfloat16)
    lo = (w - hi.astype(jnp.float32)).astype(jnp.bfloat16)
    return jnp.concatenate([hi, lo], axis=1)


def kernel(x, z, W, b):
    n_tokens, dx = x.shape
    dz = z.shape[1]
    num_experts = W.shape[1]
    w1 = _split_cat(W[:dx])
    w2 = _split_cat(W[dx:])
    b2 = b.reshape(1, num_experts)

    block = 1024
    grid = (n_tokens // block,)

    return pl.pallas_call(
        _gate_kernel,
        grid=grid,
        in_specs=[
            pl.BlockSpec((block, dx), lambda i: (i, 0)),
            pl.BlockSpec((block, dz), lambda i: (i, 0)),
            pl.BlockSpec((dx, 2 * num_experts), lambda i: (0, 0)),
            pl.BlockSpec((dz, 2 * num_experts), lambda i: (0, 0)),
            pl.BlockSpec((1, num_experts), lambda i: (0, 0)),
        ],
        out_specs=pl.BlockSpec((block, num_experts), lambda i: (i, 0)),
        out_shape=jax.ShapeDtypeStruct((n_tokens, num_experts), jnp.float32),
        compiler_params=pltpu.CompilerParams(
            dimension_semantics=("parallel",),
        ),
    )(x, z, w1, w2, b2)
